# baseline (speedup 1.0000x reference)
import jax
import jax.numpy as jnp
from jax.experimental import pallas as pl
from jax.experimental.pallas import tpu as pltpu

_C = 128
_G = 16
_VT = 3200
_NORM_EPS = 1e-12
_LN_EPS = 1e-5


def _f32dot(a, b, dims):
    return jax.lax.dot_general(a, b, (dims, ((), ())),
                               preferred_element_type=jnp.float32)


def _bdot(a, b, dims):
    return jax.lax.dot_general(a.astype(jnp.bfloat16), b.astype(jnp.bfloat16),
                               (dims, ((), ())),
                               preferred_element_type=jnp.float32)


def _make_body(nc, nv):
    def body(e_cur_ref, e_nxt_ref, w1_ref, b1_ref, w2_ref, b2_ref,
             g_ref, bb_ref, kp_ref, rpw_ref, rpb_ref, ow_ref, ob_ref,
             out_ref, kbuf_ref, rr_ref, *m_refs):
        c = pl.program_id(1)

        def ffn(e):
            z = jnp.maximum(_bdot(e, w1_ref[...], ((1,), (0,)))
                            + b1_ref[...], 0.0)
            ff = _bdot(z, w2_ref[...], ((1,), (0,))) + b2_ref[...]
            x = e + ff
            mu = jnp.mean(x, axis=1, keepdims=True)
            xc = x - mu
            var = jnp.mean(xc * xc, axis=1, keepdims=True)
            h = xc * jax.lax.rsqrt(var + _LN_EPS) * g_ref[...] + bb_ref[...]
            return _bdot(h, kp_ref[...], ((1,), (0,)))

        @pl.when(c == 0)
        def _():
            kbuf_ref[0] = ffn(e_cur_ref[0, :, 0].reshape(_G * _C, -1))
            for m_ref in m_refs:
                m_ref[...] = jnp.zeros_like(m_ref)

        @pl.when(c < nc)
        def _():
            kblk = kbuf_ref[jax.lax.rem(c, 2)]
            kbuf_ref[jax.lax.rem(c + 1, 2)] = \
                ffn(e_nxt_ref[0, :, 0].reshape(_G * _C, -1))

            row = jax.lax.broadcasted_iota(jnp.int32, (_C, 1), 0)
            valid = jnp.logical_or(c < nc - 1, row < _C - 1)

            ri = jax.lax.broadcasted_iota(jnp.int32, (_C, _C), 0)
            ci = jax.lax.broadcasted_iota(jnp.int32, (_C, _C), 1)
            eye = jnp.where(ri == ci, 1.0, 0.0)

            g_rng = range(_G)
            k_raws = [kblk[gi * _C:(gi + 1) * _C, :] for gi in g_rng]
            kms = [jnp.where(valid, k, 0.0) for k in k_raws]
            nrms = [jnp.sqrt(jnp.sum(km * km, axis=1, keepdims=True))
                    for km in kms]
            wns = [km / jnp.maximum(n, _NORM_EPS) for km, n in zip(kms, nrms)]
            wnbs = [wn.astype(jnp.bfloat16) for wn in wns]

            ss = [jax.lax.dot_general(wb, wb, ((((1,), (1,))), ((), ())),
                                      preferred_element_type=jnp.float32)
                  for wb in wnbs]
            abs_ = [jnp.where(ri > ci, s, 0.0).astype(jnp.bfloat16)
                    for s in ss]

            ts = [eye - ab.astype(jnp.float32) for ab in abs_]
            for _ in range(6):
                tbs = [t.astype(jnp.bfloat16) for t in ts]
                ats = [_bdot(ab, tb, ((1,), (0,)))
                       for ab, tb in zip(abs_, tbs)]
                resids = [(eye - t - at).astype(jnp.bfloat16)
                          for t, at in zip(ts, ats)]
                ts = [t + _bdot(tb, rs_, ((1,), (0,)))
                      for t, tb, rs_ in zip(ts, tbs, resids)]

            ms = [m_ref[...] for m_ref in m_refs]
            rhss = [km - _bdot(wb, m, ((1,), (1,)))
                    for km, wb, m in zip(kms, wnbs, ms)]
            us = [_bdot(t, rhs, ((1,), (0,))) for t, rhs in zip(ts, rhss)]
            m_news = [m + _bdot(u, wb, ((0,), (0,)))
                      for m, u, wb in zip(ms, us, wnbs)]
            for gi in g_rng:
                m_refs[gi][...] = m_news[gi]

            @pl.when(c == nc - 1)
            def _():
                rs = []
                for gi in g_rng:
                    q = k_raws[gi][_C - 1:_C, :]
                    rs.append(_f32dot(q, m_news[gi], ((1,), (1,))))
                r = jnp.concatenate(rs, axis=0)
                rr_ref[...] = jnp.dot(r, rpw_ref[...],
                                      preferred_element_type=jnp.float32) \
                    + rpb_ref[...]

        @pl.when(c >= nc)
        def _():
            out_ref[...] = jnp.dot(rr_ref[...], ow_ref[...],
                                   preferred_element_type=jnp.float32) \
                + ob_ref[...]

    return body


def kernel(seq, embed, w1, b1, w2, b2, ln_g, ln_b, kp_w, rp_w, rp_b,
           out_w, out_b):
    bsz, slen = seq.shape
    vocab, hdim = embed.shape
    hid2 = w1.shape[1]
    ng = bsz // _G
    nc = slen // _C
    nv = vocab // _VT

    e = embed[jnp.reshape(seq, (-1,))]
    es = e.reshape(ng, _G, nc, _C, hdim)

    full = lambda shape: pl.BlockSpec(shape, lambda g, c: (0, 0))
    eblk = (1, _G, 1, _C, hdim)
    vtile = lambda g, c: (0, jnp.clip(c - nc, 0, nv - 1))
    out = pl.pallas_call(
        _make_body(nc, nv),
        grid=(ng, nc + nv),
        in_specs=[
            pl.BlockSpec(eblk,
                         lambda g, c: (g, 0, jnp.minimum(c, nc - 1), 0, 0)),
            pl.BlockSpec(eblk,
                         lambda g, c: (g, 0, jnp.minimum(c + 1, nc - 1),
                                       0, 0)),
            full((hdim, hid2)), full((1, hid2)),
            full((hid2, hdim)), full((1, hdim)),
            full((1, hdim)), full((1, hdim)),
            full((hdim, hdim)),
            full((hdim, hdim)), full((1, hdim)),
            pl.BlockSpec((hdim, _VT), vtile),
            pl.BlockSpec((1, _VT), vtile),
        ],
        out_specs=pl.BlockSpec((bsz, _VT), vtile),
        out_shape=jax.ShapeDtypeStruct((bsz, vocab), jnp.float32),
        scratch_shapes=[pltpu.VMEM((2, _G * _C, hdim), jnp.float32),
                        pltpu.VMEM((bsz, hdim), jnp.float32)]
        + [pltpu.VMEM((hdim, hdim), jnp.float32) for _ in range(_G)],
        compiler_params=pltpu.CompilerParams(
            dimension_semantics=("parallel", "arbitrary"),
            vmem_limit_bytes=48 * 1024 * 1024),
    )(es, es, w1, b1.reshape(1, -1), w2, b2.reshape(1, -1),
      ln_g.reshape(1, -1), ln_b.reshape(1, -1), kp_w,
      rp_w, rp_b.reshape(1, -1), out_w, out_b.reshape(1, -1))
    return out


# speedup vs baseline: 1.0441x; 1.0441x over previous
import jax
import jax.numpy as jnp
from jax.experimental import pallas as pl
from jax.experimental.pallas import tpu as pltpu

_C = 128
_G = 16
_VT = 3200
_NORM_EPS = 1e-12
_LN_EPS = 1e-5


def _f32dot(a, b, dims):
    return jax.lax.dot_general(a, b, (dims, ((), ())),
                               preferred_element_type=jnp.float32)


def _bdot(a, b, dims):
    return jax.lax.dot_general(a.astype(jnp.bfloat16), b.astype(jnp.bfloat16),
                               (dims, ((), ())),
                               preferred_element_type=jnp.float32)


def _make_body(nc, nv):
    def body(e_cur_ref, e_nxt_ref, w1_ref, b1_ref, w2_ref, b2_ref,
             g_ref, bb_ref, kp_ref, rpw_ref, rpb_ref, ow_ref, ob_ref,
             out_ref, kbuf_ref, rr_ref, *m_refs):
        c = pl.program_id(1)

        def ffn(e):
            z = jnp.maximum(_bdot(e, w1_ref[...], ((1,), (0,)))
                            + b1_ref[...], 0.0)
            ff = _bdot(z, w2_ref[...], ((1,), (0,))) + b2_ref[...]
            x = e + ff
            mu = jnp.mean(x, axis=1, keepdims=True)
            xc = x - mu
            var = jnp.mean(xc * xc, axis=1, keepdims=True)
            h = xc * jax.lax.rsqrt(var + _LN_EPS) * g_ref[...] + bb_ref[...]
            return _bdot(h, kp_ref[...], ((1,), (0,)))

        @pl.when(c == 0)
        def _():
            kbuf_ref[0] = ffn(e_cur_ref[0, :, 0].reshape(_G * _C, -1))
            for m_ref in m_refs:
                m_ref[...] = jnp.zeros_like(m_ref)

        @pl.when(c < nc - 1)
        def _():
            kbuf_ref[jax.lax.rem(c + 1, 2)] = \
                ffn(e_nxt_ref[0, :, 0].reshape(_G * _C, -1))

        @pl.when(c < nc)
        def _():
            kblk = kbuf_ref[jax.lax.rem(c, 2)]

            row = jax.lax.broadcasted_iota(jnp.int32, (_C, 1), 0)
            valid = jnp.logical_or(c < nc - 1, row < _C - 1)

            ri = jax.lax.broadcasted_iota(jnp.int32, (_C, _C), 0)
            ci = jax.lax.broadcasted_iota(jnp.int32, (_C, _C), 1)
            eye = jnp.where(ri == ci, 1.0, 0.0)

            g_rng = range(_G)
            k_raws = [kblk[gi * _C:(gi + 1) * _C, :] for gi in g_rng]
            kms = [jnp.where(valid, k, 0.0) for k in k_raws]
            nrms = [jnp.sqrt(jnp.sum(km * km, axis=1, keepdims=True))
                    for km in kms]
            wns = [km / jnp.maximum(n, _NORM_EPS) for km, n in zip(kms, nrms)]
            wnbs = [wn.astype(jnp.bfloat16) for wn in wns]

            ss = [jax.lax.dot_general(wb, wb, ((((1,), (1,))), ((), ())),
                                      preferred_element_type=jnp.float32)
                  for wb in wnbs]
            abs_ = [jnp.where(ri > ci, s, 0.0).astype(jnp.bfloat16)
                    for s in ss]

            ts = [eye - ab.astype(jnp.float32) for ab in abs_]
            for _ in range(6):
                tbs = [t.astype(jnp.bfloat16) for t in ts]
                ats = [_bdot(ab, tb, ((1,), (0,)))
                       for ab, tb in zip(abs_, tbs)]
                resids = [(eye - t - at).astype(jnp.bfloat16)
                          for t, at in zip(ts, ats)]
                ts = [t + _bdot(tb, rs_, ((1,), (0,)))
                      for t, tb, rs_ in zip(ts, tbs, resids)]

            ms = [m_ref[...] for m_ref in m_refs]
            rhss = [km - _bdot(wb, m, ((1,), (1,)))
                    for km, wb, m in zip(kms, wnbs, ms)]
            us = [_bdot(t, rhs, ((1,), (0,))) for t, rhs in zip(ts, rhss)]
            m_news = [m + _bdot(u, wb, ((0,), (0,)))
                      for m, u, wb in zip(ms, us, wnbs)]
            for gi in g_rng:
                m_refs[gi][...] = m_news[gi]

            @pl.when(c == nc - 1)
            def _():
                rs = []
                for gi in g_rng:
                    q = k_raws[gi][_C - 1:_C, :]
                    rs.append(_f32dot(q, m_news[gi], ((1,), (1,))))
                r = jnp.concatenate(rs, axis=0)
                rr_ref[...] = jnp.dot(r, rpw_ref[...],
                                      preferred_element_type=jnp.float32) \
                    + rpb_ref[...]

        @pl.when(c >= nc)
        def _():
            out_ref[...] = jnp.dot(rr_ref[...], ow_ref[...],
                                   preferred_element_type=jnp.float32) \
                + ob_ref[...]

    return body


def kernel(seq, embed, w1, b1, w2, b2, ln_g, ln_b, kp_w, rp_w, rp_b,
           out_w, out_b):
    bsz, slen = seq.shape
    vocab, hdim = embed.shape
    hid2 = w1.shape[1]
    ng = bsz // _G
    nc = slen // _C
    nv = vocab // _VT

    e = embed[jnp.reshape(seq, (-1,))]
    es = e.reshape(ng, _G, nc, _C, hdim)

    full = lambda shape: pl.BlockSpec(shape, lambda g, c: (0, 0))
    eblk = (1, _G, 1, _C, hdim)
    vtile = lambda g, c: (0, jnp.clip(c - nc, 0, nv - 1))
    out = pl.pallas_call(
        _make_body(nc, nv),
        grid=(ng, nc + nv),
        in_specs=[
            pl.BlockSpec(eblk,
                         lambda g, c: (g, 0, jnp.minimum(c, nc - 1), 0, 0)),
            pl.BlockSpec(eblk,
                         lambda g, c: (g, 0, jnp.minimum(c + 1, nc - 1),
                                       0, 0)),
            full((hdim, hid2)), full((1, hid2)),
            full((hid2, hdim)), full((1, hdim)),
            full((1, hdim)), full((1, hdim)),
            full((hdim, hdim)),
            full((hdim, hdim)), full((1, hdim)),
            pl.BlockSpec((hdim, _VT), vtile),
            pl.BlockSpec((1, _VT), vtile),
        ],
        out_specs=pl.BlockSpec((bsz, _VT), vtile),
        out_shape=jax.ShapeDtypeStruct((bsz, vocab), jnp.float32),
        scratch_shapes=[pltpu.VMEM((2, _G * _C, hdim), jnp.float32),
                        pltpu.VMEM((bsz, hdim), jnp.float32)]
        + [pltpu.VMEM((hdim, hdim), jnp.float32) for _ in range(_G)],
        compiler_params=pltpu.CompilerParams(
            dimension_semantics=("parallel", "arbitrary"),
            vmem_limit_bytes=48 * 1024 * 1024),
    )(es, es, w1, b1.reshape(1, -1), w2, b2.reshape(1, -1),
      ln_g.reshape(1, -1), ln_b.reshape(1, -1), kp_w,
      rp_w, rp_b.reshape(1, -1), out_w, out_b.reshape(1, -1))
    return out
